# baseline (speedup 1.0000x reference)
import jax
import jax.numpy as jnp
from jax.experimental import pallas as pl
from jax.experimental.pallas import tpu as pltpu

NCLS = 24
LANES = 128
S0 = 256
S1 = 256


def _kl(x, y):
    mx = jnp.max(x, axis=0, keepdims=True)
    ex = jnp.exp(x - mx)
    sx = jnp.sum(ex, axis=0, keepdims=True)
    my = jnp.max(y, axis=0, keepdims=True)
    ey = jnp.exp(y - my)
    sy = jnp.sum(ey, axis=0, keepdims=True)
    t = jnp.sum(ex * (x - y), axis=0, keepdims=True) / sx
    return t - (mx + jnp.log(sx)) + (my + jnp.log(sy))


def _bin(kl, lab, sums_ref, cnts_ref):
    s = kl.shape[1]
    cls = jax.lax.broadcasted_iota(jnp.int32, (NCLS, 1), 0)
    mask = lab == cls
    contrib = jnp.where(mask, kl, jnp.float32(0.0))
    cnt = mask.astype(jnp.float32)
    part_s = jnp.zeros((NCLS, LANES), jnp.float32)
    part_c = jnp.zeros((NCLS, LANES), jnp.float32)
    for j in range(s // LANES):
        part_s = part_s + contrib[:, j * LANES:(j + 1) * LANES]
        part_c = part_c + cnt[:, j * LANES:(j + 1) * LANES]
    sums_ref[...] += part_s
    cnts_ref[...] += part_c


def _scale0_body(x_ref, y_ref, lab_ref, sums_ref, cnts_ref):
    i = pl.program_id(0)

    @pl.when(i == 0)
    def _init():
        sums_ref[...] = jnp.zeros_like(sums_ref)
        cnts_ref[...] = jnp.zeros_like(cnts_ref)

    _bin(_kl(x_ref[0], y_ref[0]), lab_ref[0], sums_ref, cnts_ref)


def _scale1_body(gate_ref, x_ref, y_ref, lab_ref, s0_ref, c0_ref,
                 out_ref, s1_ref, c1_ref):
    i = pl.program_id(0)

    @pl.when(i == 0)
    def _init():
        s1_ref[...] = jnp.zeros_like(s1_ref)
        c1_ref[...] = jnp.zeros_like(c1_ref)

    _bin(_kl(x_ref[0], y_ref[0]), lab_ref[0], s1_ref, c1_ref)

    @pl.when(i == pl.num_programs(0) - 1)
    def _combine():
        gate = gate_ref[:, :1]

        def term(s, c):
            sc = jnp.sum(s, axis=1, keepdims=True)
            cc = jnp.sum(c, axis=1, keepdims=True)
            klc = sc / jnp.maximum(cc, 1.0)
            return jnp.sum(gate * jnp.where(cc > 0, klc, jnp.float32(0.0)))

        loss = term(s0_ref[...], c0_ref[...]) + jnp.float32(2.0) * term(s1_ref[...], c1_ref[...])
        out_ref[...] = jnp.full((8, LANES), loss, jnp.float32)


def kernel(pseudo_labels, feat_old_0, feat_0, feat_old_1, feat_1, num_class, num_old_class):
    b = pseudo_labels.shape[0]

    lab0 = pseudo_labels[:, 0, ::8, ::8].reshape(b * 4096 // S0, 1, S0)
    lab1 = pseudo_labels[:, 0, ::16, ::16].reshape(b * 1024 // S1, 1, S1)

    x0 = feat_0.reshape(b, 384, 4096)
    y0 = feat_old_0.reshape(b, 384, 4096)
    x1 = feat_1.reshape(b, 768, 1024)
    y1 = feat_old_1.reshape(b, 768, 1024)

    cls = jnp.arange(NCLS, dtype=jnp.float32)
    noc = jnp.asarray(num_old_class, jnp.float32)
    nc = jnp.asarray(num_class, jnp.float32)
    gate = jnp.where(
        cls == 0,
        noc / nc,
        jnp.where((cls <= noc) & (cls < 21), jnp.float32(1.0), jnp.float32(0.0)),
    )
    gate2d = jnp.broadcast_to(gate[:, None], (NCLS, LANES))

    acc_spec = pl.BlockSpec((NCLS, LANES), lambda i: (0, 0))
    nb0 = 4096 // S0
    f0 = pl.BlockSpec((1, 384, S0), lambda i: (i // nb0, 0, i % nb0))
    s0, c0 = pl.pallas_call(
        _scale0_body,
        grid=(b * nb0,),
        in_specs=[f0, f0, pl.BlockSpec((1, 1, S0), lambda i: (i, 0, 0))],
        out_specs=[acc_spec, acc_spec],
        out_shape=[jax.ShapeDtypeStruct((NCLS, LANES), jnp.float32)] * 2,
    )(x0, y0, lab0)

    nb1 = 1024 // S1
    f1 = pl.BlockSpec((1, 768, S1), lambda i: (i // nb1, 0, i % nb1))
    out = pl.pallas_call(
        _scale1_body,
        grid=(b * nb1,),
        in_specs=[acc_spec, f1, f1,
                  pl.BlockSpec((1, 1, S1), lambda i: (i, 0, 0)),
                  acc_spec, acc_spec],
        out_specs=pl.BlockSpec((8, LANES), lambda i: (0, 0)),
        out_shape=jax.ShapeDtypeStruct((8, LANES), jnp.float32),
        scratch_shapes=[pltpu.VMEM((NCLS, LANES), jnp.float32)] * 2,
    )(gate2d, x1, y1, lab1, s0, c0)
    return out[0, 0]


# speedup vs baseline: 1.1910x; 1.1910x over previous
import jax
import jax.numpy as jnp
from jax.experimental import pallas as pl
from jax.experimental.pallas import tpu as pltpu

NCLS = 24
LANES = 128
S0 = 1024
S1 = 512


def _kl(x, y):
    mx = jnp.max(x, axis=0, keepdims=True)
    ex = jnp.exp(x - mx)
    sx = jnp.sum(ex, axis=0, keepdims=True)
    my = jnp.max(y, axis=0, keepdims=True)
    ey = jnp.exp(y - my)
    sy = jnp.sum(ey, axis=0, keepdims=True)
    t = jnp.sum(ex * (x - y), axis=0, keepdims=True) / sx
    return t - (mx + jnp.log(sx)) + (my + jnp.log(sy))


def _bin(kl, lab, sums_ref, cnts_ref):
    s = kl.shape[1]
    cls = jax.lax.broadcasted_iota(jnp.int32, (NCLS, 1), 0)
    mask = lab == cls
    contrib = jnp.where(mask, kl, jnp.float32(0.0))
    cnt = mask.astype(jnp.float32)
    part_s = jnp.zeros((NCLS, LANES), jnp.float32)
    part_c = jnp.zeros((NCLS, LANES), jnp.float32)
    for j in range(s // LANES):
        part_s = part_s + contrib[:, j * LANES:(j + 1) * LANES]
        part_c = part_c + cnt[:, j * LANES:(j + 1) * LANES]
    sums_ref[...] += part_s
    cnts_ref[...] += part_c


def _scale0_body(x_ref, y_ref, lab_ref, sums_ref, cnts_ref):
    i = pl.program_id(0)

    @pl.when(i == 0)
    def _init():
        sums_ref[...] = jnp.zeros_like(sums_ref)
        cnts_ref[...] = jnp.zeros_like(cnts_ref)

    _bin(_kl(x_ref[0], y_ref[0]), lab_ref[0], sums_ref, cnts_ref)


def _scale1_body(gate_ref, x_ref, y_ref, lab_ref, s0_ref, c0_ref,
                 out_ref, s1_ref, c1_ref):
    i = pl.program_id(0)

    @pl.when(i == 0)
    def _init():
        s1_ref[...] = jnp.zeros_like(s1_ref)
        c1_ref[...] = jnp.zeros_like(c1_ref)

    _bin(_kl(x_ref[0], y_ref[0]), lab_ref[0], s1_ref, c1_ref)

    @pl.when(i == pl.num_programs(0) - 1)
    def _combine():
        gate = gate_ref[:, :1]

        def term(s, c):
            sc = jnp.sum(s, axis=1, keepdims=True)
            cc = jnp.sum(c, axis=1, keepdims=True)
            klc = sc / jnp.maximum(cc, 1.0)
            return jnp.sum(gate * jnp.where(cc > 0, klc, jnp.float32(0.0)))

        loss = term(s0_ref[...], c0_ref[...]) + jnp.float32(2.0) * term(s1_ref[...], c1_ref[...])
        out_ref[...] = jnp.full((8, LANES), loss, jnp.float32)


def kernel(pseudo_labels, feat_old_0, feat_0, feat_old_1, feat_1, num_class, num_old_class):
    b = pseudo_labels.shape[0]

    lab0 = pseudo_labels[:, 0, ::8, ::8].reshape(b * 4096 // S0, 1, S0)
    lab1 = pseudo_labels[:, 0, ::16, ::16].reshape(b * 1024 // S1, 1, S1)

    x0 = feat_0.reshape(b, 384, 4096)
    y0 = feat_old_0.reshape(b, 384, 4096)
    x1 = feat_1.reshape(b, 768, 1024)
    y1 = feat_old_1.reshape(b, 768, 1024)

    cls = jnp.arange(NCLS, dtype=jnp.float32)
    noc = jnp.asarray(num_old_class, jnp.float32)
    nc = jnp.asarray(num_class, jnp.float32)
    gate = jnp.where(
        cls == 0,
        noc / nc,
        jnp.where((cls <= noc) & (cls < 21), jnp.float32(1.0), jnp.float32(0.0)),
    )
    gate2d = jnp.broadcast_to(gate[:, None], (NCLS, LANES))

    acc_spec = pl.BlockSpec((NCLS, LANES), lambda i: (0, 0))
    nb0 = 4096 // S0
    f0 = pl.BlockSpec((1, 384, S0), lambda i: (i // nb0, 0, i % nb0))
    s0, c0 = pl.pallas_call(
        _scale0_body,
        grid=(b * nb0,),
        in_specs=[f0, f0, pl.BlockSpec((1, 1, S0), lambda i: (i, 0, 0))],
        out_specs=[acc_spec, acc_spec],
        out_shape=[jax.ShapeDtypeStruct((NCLS, LANES), jnp.float32)] * 2,
    )(x0, y0, lab0)

    nb1 = 1024 // S1
    f1 = pl.BlockSpec((1, 768, S1), lambda i: (i // nb1, 0, i % nb1))
    out = pl.pallas_call(
        _scale1_body,
        grid=(b * nb1,),
        in_specs=[acc_spec, f1, f1,
                  pl.BlockSpec((1, 1, S1), lambda i: (i, 0, 0)),
                  acc_spec, acc_spec],
        out_specs=pl.BlockSpec((8, LANES), lambda i: (0, 0)),
        out_shape=jax.ShapeDtypeStruct((8, LANES), jnp.float32),
        scratch_shapes=[pltpu.VMEM((NCLS, LANES), jnp.float32)] * 2,
    )(gate2d, x1, y1, lab1, s0, c0)
    return out[0, 0]


# speedup vs baseline: 1.1920x; 1.0008x over previous
import jax
import jax.numpy as jnp
from jax.experimental import pallas as pl
from jax.experimental.pallas import tpu as pltpu

NCLS = 24
LANES = 128
S0 = 1024
S1 = 1024


def _kl(x, y):
    mx = jnp.max(x, axis=0, keepdims=True)
    ex = jnp.exp(x - mx)
    sx = jnp.sum(ex, axis=0, keepdims=True)
    my = jnp.max(y, axis=0, keepdims=True)
    ey = jnp.exp(y - my)
    sy = jnp.sum(ey, axis=0, keepdims=True)
    t = jnp.sum(ex * (x - y), axis=0, keepdims=True) / sx
    return t - (mx + jnp.log(sx)) + (my + jnp.log(sy))


def _bin(kl, lab, sums_ref, cnts_ref):
    s = kl.shape[1]
    cls = jax.lax.broadcasted_iota(jnp.int32, (NCLS, 1), 0)
    mask = lab == cls
    contrib = jnp.where(mask, kl, jnp.float32(0.0))
    cnt = mask.astype(jnp.float32)
    part_s = jnp.zeros((NCLS, LANES), jnp.float32)
    part_c = jnp.zeros((NCLS, LANES), jnp.float32)
    for j in range(s // LANES):
        part_s = part_s + contrib[:, j * LANES:(j + 1) * LANES]
        part_c = part_c + cnt[:, j * LANES:(j + 1) * LANES]
    sums_ref[...] += part_s
    cnts_ref[...] += part_c


def _scale0_body(x_ref, y_ref, lab_ref, sums_ref, cnts_ref):
    i = pl.program_id(0)

    @pl.when(i == 0)
    def _init():
        sums_ref[...] = jnp.zeros_like(sums_ref)
        cnts_ref[...] = jnp.zeros_like(cnts_ref)

    _bin(_kl(x_ref[0], y_ref[0]), lab_ref[0], sums_ref, cnts_ref)


def _scale1_body(gate_ref, x_ref, y_ref, lab_ref, s0_ref, c0_ref,
                 out_ref, s1_ref, c1_ref):
    i = pl.program_id(0)

    @pl.when(i == 0)
    def _init():
        s1_ref[...] = jnp.zeros_like(s1_ref)
        c1_ref[...] = jnp.zeros_like(c1_ref)

    _bin(_kl(x_ref[0], y_ref[0]), lab_ref[0], s1_ref, c1_ref)

    @pl.when(i == pl.num_programs(0) - 1)
    def _combine():
        gate = gate_ref[:, :1]

        def term(s, c):
            sc = jnp.sum(s, axis=1, keepdims=True)
            cc = jnp.sum(c, axis=1, keepdims=True)
            klc = sc / jnp.maximum(cc, 1.0)
            return jnp.sum(gate * jnp.where(cc > 0, klc, jnp.float32(0.0)))

        loss = term(s0_ref[...], c0_ref[...]) + jnp.float32(2.0) * term(s1_ref[...], c1_ref[...])
        out_ref[...] = jnp.full((8, LANES), loss, jnp.float32)


def kernel(pseudo_labels, feat_old_0, feat_0, feat_old_1, feat_1, num_class, num_old_class):
    b = pseudo_labels.shape[0]

    lab0 = pseudo_labels[:, 0, ::8, ::8].reshape(b * 4096 // S0, 1, S0)
    lab1 = pseudo_labels[:, 0, ::16, ::16].reshape(b * 1024 // S1, 1, S1)

    x0 = feat_0.reshape(b, 384, 4096)
    y0 = feat_old_0.reshape(b, 384, 4096)
    x1 = feat_1.reshape(b, 768, 1024)
    y1 = feat_old_1.reshape(b, 768, 1024)

    cls = jnp.arange(NCLS, dtype=jnp.float32)
    noc = jnp.asarray(num_old_class, jnp.float32)
    nc = jnp.asarray(num_class, jnp.float32)
    gate = jnp.where(
        cls == 0,
        noc / nc,
        jnp.where((cls <= noc) & (cls < 21), jnp.float32(1.0), jnp.float32(0.0)),
    )
    gate2d = jnp.broadcast_to(gate[:, None], (NCLS, LANES))

    acc_spec = pl.BlockSpec((NCLS, LANES), lambda i: (0, 0))
    nb0 = 4096 // S0
    f0 = pl.BlockSpec((1, 384, S0), lambda i: (i // nb0, 0, i % nb0))
    s0, c0 = pl.pallas_call(
        _scale0_body,
        grid=(b * nb0,),
        in_specs=[f0, f0, pl.BlockSpec((1, 1, S0), lambda i: (i, 0, 0))],
        out_specs=[acc_spec, acc_spec],
        out_shape=[jax.ShapeDtypeStruct((NCLS, LANES), jnp.float32)] * 2,
    )(x0, y0, lab0)

    nb1 = 1024 // S1
    f1 = pl.BlockSpec((1, 768, S1), lambda i: (i // nb1, 0, i % nb1))
    out = pl.pallas_call(
        _scale1_body,
        grid=(b * nb1,),
        in_specs=[acc_spec, f1, f1,
                  pl.BlockSpec((1, 1, S1), lambda i: (i, 0, 0)),
                  acc_spec, acc_spec],
        out_specs=pl.BlockSpec((8, LANES), lambda i: (0, 0)),
        out_shape=jax.ShapeDtypeStruct((8, LANES), jnp.float32),
        scratch_shapes=[pltpu.VMEM((NCLS, LANES), jnp.float32)] * 2,
    )(gate2d, x1, y1, lab1, s0, c0)
    return out[0, 0]
